# baseline (speedup 1.0000x reference)
import jax
import jax.numpy as jnp
from jax.experimental import pallas as pl
from jax.experimental.pallas import tpu as pltpu

_LEAKY = 0.1
_VMEM_LIMIT = 48 * 1024 * 1024


def _make_level_body(HW, W, has_up):
    pad = W
    rows_p = pad + HW + W

    def body(*refs):
        it = iter(refs)
        xa_ref = next(it)
        if has_up:
            xlo_ref = next(it)
            u_ref = next(it)
            wup_ref = next(it)
            bup_ref = next(it)
            w0b_ref = next(it)
        w0a_ref = next(it)
        b0_ref = next(it)
        w1_ref = next(it)
        b1_ref = next(it)
        w2_ref = next(it)
        b2_ref = next(it)
        w3_ref = next(it)
        b3_ref = next(it)
        w4_ref = next(it)
        b4t_ref = next(it)
        o_ref = next(it)
        p_ref = next(it)

        def leaky(v):
            return jnp.maximum(v, _LEAKY * v)

        def mm(a, w):
            return jnp.dot(a.astype(jnp.bfloat16), w,
                           preferred_element_type=jnp.float32)

        def mm_ta(a_cm, w):
            return jax.lax.dot_general(
                a_cm.astype(jnp.bfloat16), w,
                dimension_numbers=(((0,), (0,)), ((), ())),
                preferred_element_type=jnp.float32)

        acc0 = mm_ta(xa_ref[0], w0a_ref[...])
        if has_up:
            zu = leaky(mm_ta(xlo_ref[0], wup_ref[...]) + bup_ref[...])
            zlo = mm(zu, w0b_ref[...])
            acc0 = acc0 + jnp.dot(u_ref[...], zlo.astype(jnp.bfloat16),
                                  preferred_element_type=jnp.float32)
        act = leaky(acc0 + b0_ref[...])

        col = jax.lax.broadcasted_iota(jnp.int32, (HW, 1), 0) % W
        not_l = col != 0
        not_r = col != (W - 1)

        C3 = p_ref.shape[1]
        C = C3 // 3

        p_ref[0:pad, :] = jnp.zeros((pad, C3), jnp.bfloat16)
        p_ref[pad + HW:rows_p, :] = jnp.zeros((rows_p - pad - HW, C3), jnp.bfloat16)
        p_ref[pad:pad + 1, 0:C] = jnp.zeros((1, C), jnp.bfloat16)
        p_ref[pad + HW - 1:pad + HW, 2 * C:C3] = jnp.zeros((1, C), jnp.bfloat16)

        def conv3(a, w_ref, b_ref):
            ab = a.astype(jnp.bfloat16)
            p_ref[pad + 1:pad + 1 + HW, 0:C] = jnp.where(not_r, ab, 0)
            p_ref[pad:pad + HW, C:2 * C] = ab
            p_ref[pad - 1:pad - 1 + HW, 2 * C:C3] = jnp.where(not_l, ab, 0)
            cout = w_ref.shape[2]
            acc = jnp.zeros((HW, cout), jnp.float32)
            for dy in range(3):
                base = pad + (dy - 1) * W
                acc = acc + jnp.dot(p_ref[base:base + HW, :], w_ref[dy],
                                    preferred_element_type=jnp.float32)
            return leaky(acc + b_ref[...])

        z = conv3(act, w1_ref, b1_ref)
        z = leaky(mm(z, w2_ref[...]) + b2_ref[...])
        z = conv3(z, w3_ref, b3_ref)
        zt = jax.lax.dot_general(
            w4_ref[...], z.astype(jnp.bfloat16),
            dimension_numbers=(((0,), (1,)), ((), ())),
            preferred_element_type=jnp.float32)
        o_ref[0] = leaky(zt + b4t_ref[...])

    return body


def _run_level(xcm, layers, H, W, up=None):
    N, Ca, HW = xcm.shape
    C = layers[4][0].shape[-1]
    has_up = up is not None
    pad = W
    rows_p = pad + HW + W

    def const_spec(shape):
        nd = len(shape)
        return pl.BlockSpec(shape, lambda n, _nd=nd: (0,) * _nd)

    inputs = [xcm]
    in_specs = [pl.BlockSpec((1, Ca, HW), lambda n: (n, 0, 0))]

    if has_up:
        xlo, U = up["xlo"], up["U"]
        Clo, hw = xlo.shape[1], xlo.shape[2]
        w0 = layers[0][0]
        w0a, w0b = w0[:Ca], w0[Ca:]
        inputs += [xlo, U, up["w"], up["b"], w0b]
        in_specs += [pl.BlockSpec((1, Clo, hw), lambda n: (n, 0, 0)),
                     const_spec(U.shape), const_spec(up["w"].shape),
                     const_spec(up["b"].shape), const_spec(w0b.shape)]
    else:
        w0a = layers[0][0]

    inputs += [w0a, layers[0][1]]
    in_specs += [const_spec(w0a.shape), const_spec(layers[0][1].shape)]
    for w, b in layers[1:]:
        inputs += [w, b]
        in_specs += [const_spec(w.shape), const_spec(b.shape)]

    flops = 2 * HW * (Ca * C + 2 * (9 * C * 2 * C) + 2 * (2 * C * C))
    if has_up:
        flops += 2 * (hw * Clo * C + hw * C * C + HW * hw * C)
    flops *= N
    bytes_accessed = (sum(int(a.size) * a.dtype.itemsize for a in inputs)
                      + N * HW * C * 4)

    return pl.pallas_call(
        _make_level_body(HW, W, has_up),
        out_shape=jax.ShapeDtypeStruct((N, C, HW), jnp.float32),
        grid_spec=pltpu.PrefetchScalarGridSpec(
            num_scalar_prefetch=0,
            grid=(N,),
            in_specs=in_specs,
            out_specs=pl.BlockSpec((1, C, HW), lambda n: (n, 0, 0)),
            scratch_shapes=[pltpu.VMEM((rows_p, 3 * C), jnp.bfloat16)],
        ),
        compiler_params=pltpu.CompilerParams(
            dimension_semantics=("parallel",),
            vmem_limit_bytes=_VMEM_LIMIT),
        cost_estimate=pl.CostEstimate(flops=flops, transcendentals=0,
                                      bytes_accessed=bytes_accessed),
    )(*inputs)


def _up_matrix(H, W):
    h, w = H // 2, W // 2
    r = jnp.arange(H * W, dtype=jnp.int32)
    lo = (r // W // 2) * w + (r % W) // 2
    return (lo[:, None] == jnp.arange(h * w, dtype=jnp.int32)[None, :]).astype(jnp.bfloat16)


def _stack_taps(w9):
    k9, C, C2 = w9.shape
    return w9.reshape(3, 3 * C, C2)


def kernel(x0, x1, x2,
           seq1_0_w, seq1_0_b, seq1_1_w, seq1_1_b, seq1_2_w, seq1_2_b,
           seq1_3_w, seq1_3_b, seq1_4_w, seq1_4_b,
           seq2_0_w, seq2_0_b, seq2_1_w, seq2_1_b, seq2_2_w, seq2_2_b,
           seq2_3_w, seq2_3_b, seq2_4_w, seq2_4_b,
           seq3_0_w, seq3_0_b, seq3_1_w, seq3_1_b, seq3_2_w, seq3_2_b,
           seq3_3_w, seq3_3_b, seq3_4_w, seq3_4_b,
           up1_w, up1_b, up2_w, up2_b):
    seqs = {
        1: [(seq1_0_w, seq1_0_b), (_stack_taps(seq1_1_w), seq1_1_b),
            (seq1_2_w, seq1_2_b), (_stack_taps(seq1_3_w), seq1_3_b),
            (seq1_4_w, seq1_4_b.reshape(-1, 1))],
        2: [(seq2_0_w, seq2_0_b), (_stack_taps(seq2_1_w), seq2_1_b),
            (seq2_2_w, seq2_2_b), (_stack_taps(seq2_3_w), seq2_3_b),
            (seq2_4_w, seq2_4_b.reshape(-1, 1))],
        3: [(seq3_0_w, seq3_0_b), (_stack_taps(seq3_1_w), seq3_1_b),
            (seq3_2_w, seq3_2_b), (_stack_taps(seq3_3_w), seq3_3_b),
            (seq3_4_w, seq3_4_b.reshape(-1, 1))],
    }

    dims = [(x.shape[2], x.shape[3]) for x in (x0, x1, x2)]
    feats = [x.reshape(x.shape[0], x.shape[1], -1) for x in (x0, x1, x2)]
    (H0, W0), (H1, W1), (H2, W2) = dims

    n3 = _run_level(feats[2], seqs[3], H2, W2)
    n2 = _run_level(feats[1], seqs[2], H1, W1,
                    up=dict(xlo=n3, w=up2_w, b=up2_b, U=_up_matrix(H1, W1)))
    n1 = _run_level(feats[0], seqs[1], H0, W0,
                    up=dict(xlo=n2, w=up1_w, b=up1_b, U=_up_matrix(H0, W0)))

    N = x0.shape[0]
    return [n1.reshape(N, -1, H0, W0), n2.reshape(N, -1, H1, W1),
            n3.reshape(N, -1, H2, W2)]


# speedup vs baseline: 1.0425x; 1.0425x over previous
import jax
import jax.numpy as jnp
from jax.experimental import pallas as pl
from jax.experimental.pallas import tpu as pltpu

_LEAKY = 0.1
_VMEM_LIMIT = 48 * 1024 * 1024


def _make_level_body(HW, W, has_up):
    pad = W + 8
    rows_p = pad + HW + W + 8
    H = HW // W

    def body(*refs):
        it = iter(refs)
        xa_ref = next(it)
        if has_up:
            xlo_ref = next(it)
            wup_ref = next(it)
            bup_ref = next(it)
            w0b_ref = next(it)
        w0a_ref = next(it)
        b0_ref = next(it)
        w1_ref = next(it)
        b1_ref = next(it)
        w2_ref = next(it)
        b2_ref = next(it)
        w3_ref = next(it)
        b3_ref = next(it)
        w4_ref = next(it)
        b4t_ref = next(it)
        o_ref = next(it)
        p_ref = next(it)

        def leaky(v):
            return jnp.maximum(v, _LEAKY * v)

        def mm(a, w):
            return jnp.dot(a.astype(jnp.bfloat16), w,
                           preferred_element_type=jnp.float32)

        def mm_ta(a_cm, w):
            return jax.lax.dot_general(
                a_cm.astype(jnp.bfloat16), w,
                dimension_numbers=(((0,), (0,)), ((), ())),
                preferred_element_type=jnp.float32)

        acc0 = mm_ta(xa_ref[0], w0a_ref[...])
        if has_up:
            zu = leaky(mm_ta(xlo_ref[0], wup_ref[...]) + bup_ref[...])
            zlo = mm(zu, w0b_ref[...])
            h, w = H // 2, W // 2
            z3 = zlo.astype(jnp.bfloat16).astype(jnp.float32).reshape(h, w, -1)
            z3 = jnp.repeat(jnp.repeat(z3, 2, axis=1), 2, axis=0)
            acc0 = acc0 + z3.reshape(HW, -1)
        act = leaky(acc0 + b0_ref[...])

        col = jax.lax.broadcasted_iota(jnp.int32, (HW, 1), 0) % W
        not_l = col != 0
        not_r = col != (W - 1)

        C9 = p_ref.shape[1]
        C = C9 // 9

        p_ref[pad:pad + W + 1, 0:6 * C] = jnp.zeros((W + 1, 6 * C), jnp.bfloat16)
        p_ref[pad + HW - W - 1:pad + HW, 3 * C:C9] = (
            jnp.zeros((W + 1, 6 * C), jnp.bfloat16))

        def conv3(a, w_ref, b_ref):
            ab = a.astype(jnp.bfloat16)
            m = (jnp.where(not_r, ab, 0), ab, jnp.where(not_l, ab, 0))
            for dy in range(3):
                for dx in range(3):
                    ofs = pad - (dy - 1) * W + (1 - dx)
                    k = dy * 3 + dx
                    p_ref[ofs:ofs + HW, k * C:(k + 1) * C] = m[dx]
            acc = jnp.dot(p_ref[pad:pad + HW, :], w_ref[...],
                          preferred_element_type=jnp.float32)
            return leaky(acc + b_ref[...])

        z = conv3(act, w1_ref, b1_ref)
        z = leaky(mm(z, w2_ref[...]) + b2_ref[...])
        z = conv3(z, w3_ref, b3_ref)
        zt = jax.lax.dot_general(
            w4_ref[...], z.astype(jnp.bfloat16),
            dimension_numbers=(((0,), (1,)), ((), ())),
            preferred_element_type=jnp.float32)
        o_ref[0] = leaky(zt + b4t_ref[...])

    return body


def _run_level(xcm, layers, H, W, up=None):
    N, Ca, HW = xcm.shape
    C = layers[4][0].shape[-1]
    has_up = up is not None
    pad = W + 8
    rows_p = pad + HW + W + 8

    def const_spec(shape):
        nd = len(shape)
        return pl.BlockSpec(shape, lambda n, _nd=nd: (0,) * _nd)

    inputs = [xcm]
    in_specs = [pl.BlockSpec((1, Ca, HW), lambda n: (n, 0, 0))]

    if has_up:
        xlo = up["xlo"]
        Clo, hw = xlo.shape[1], xlo.shape[2]
        w0 = layers[0][0]
        w0a, w0b = w0[:Ca], w0[Ca:]
        inputs += [xlo, up["w"], up["b"], w0b]
        in_specs += [pl.BlockSpec((1, Clo, hw), lambda n: (n, 0, 0)),
                     const_spec(up["w"].shape),
                     const_spec(up["b"].shape), const_spec(w0b.shape)]
    else:
        w0a = layers[0][0]

    inputs += [w0a, layers[0][1]]
    in_specs += [const_spec(w0a.shape), const_spec(layers[0][1].shape)]
    for w, b in layers[1:]:
        inputs += [w, b]
        in_specs += [const_spec(w.shape), const_spec(b.shape)]

    flops = 2 * HW * (Ca * C + 2 * (9 * C * 2 * C) + 2 * (2 * C * C))
    if has_up:
        flops += 2 * (hw * Clo * C + hw * C * C)
    flops *= N
    bytes_accessed = (sum(int(a.size) * a.dtype.itemsize for a in inputs)
                      + N * HW * C * 4)

    return pl.pallas_call(
        _make_level_body(HW, W, has_up),
        out_shape=jax.ShapeDtypeStruct((N, C, HW), jnp.float32),
        grid_spec=pltpu.PrefetchScalarGridSpec(
            num_scalar_prefetch=0,
            grid=(N,),
            in_specs=in_specs,
            out_specs=pl.BlockSpec((1, C, HW), lambda n: (n, 0, 0)),
            scratch_shapes=[pltpu.VMEM((rows_p, 9 * C), jnp.bfloat16)],
        ),
        compiler_params=pltpu.CompilerParams(
            dimension_semantics=("parallel",),
            vmem_limit_bytes=_VMEM_LIMIT),
        cost_estimate=pl.CostEstimate(flops=flops, transcendentals=0,
                                      bytes_accessed=bytes_accessed),
    )(*inputs)


def _stack_taps(w9):
    k9, C, C2 = w9.shape
    return w9.reshape(9 * C, C2)


def kernel(x0, x1, x2,
           seq1_0_w, seq1_0_b, seq1_1_w, seq1_1_b, seq1_2_w, seq1_2_b,
           seq1_3_w, seq1_3_b, seq1_4_w, seq1_4_b,
           seq2_0_w, seq2_0_b, seq2_1_w, seq2_1_b, seq2_2_w, seq2_2_b,
           seq2_3_w, seq2_3_b, seq2_4_w, seq2_4_b,
           seq3_0_w, seq3_0_b, seq3_1_w, seq3_1_b, seq3_2_w, seq3_2_b,
           seq3_3_w, seq3_3_b, seq3_4_w, seq3_4_b,
           up1_w, up1_b, up2_w, up2_b):
    seqs = {
        1: [(seq1_0_w, seq1_0_b), (_stack_taps(seq1_1_w), seq1_1_b),
            (seq1_2_w, seq1_2_b), (_stack_taps(seq1_3_w), seq1_3_b),
            (seq1_4_w, seq1_4_b.reshape(-1, 1))],
        2: [(seq2_0_w, seq2_0_b), (_stack_taps(seq2_1_w), seq2_1_b),
            (seq2_2_w, seq2_2_b), (_stack_taps(seq2_3_w), seq2_3_b),
            (seq2_4_w, seq2_4_b.reshape(-1, 1))],
        3: [(seq3_0_w, seq3_0_b), (_stack_taps(seq3_1_w), seq3_1_b),
            (seq3_2_w, seq3_2_b), (_stack_taps(seq3_3_w), seq3_3_b),
            (seq3_4_w, seq3_4_b.reshape(-1, 1))],
    }

    dims = [(x.shape[2], x.shape[3]) for x in (x0, x1, x2)]
    feats = [x.reshape(x.shape[0], x.shape[1], -1) for x in (x0, x1, x2)]
    (H0, W0), (H1, W1), (H2, W2) = dims

    n3 = _run_level(feats[2], seqs[3], H2, W2)
    n2 = _run_level(feats[1], seqs[2], H1, W1,
                    up=dict(xlo=n3, w=up2_w, b=up2_b))
    n1 = _run_level(feats[0], seqs[1], H0, W0,
                    up=dict(xlo=n2, w=up1_w, b=up1_b))

    N = x0.shape[0]
    return [n1.reshape(N, -1, H0, W0), n2.reshape(N, -1, H1, W1),
            n3.reshape(N, -1, H2, W2)]


# speedup vs baseline: 1.2125x; 1.1631x over previous
import jax
import jax.numpy as jnp
from jax.experimental import pallas as pl
from jax.experimental.pallas import tpu as pltpu

_LEAKY = 0.1
_VMEM_LIMIT = 48 * 1024 * 1024


def _make_level_body(HW, W, has_up):
    pad = W + 8
    rows_p = pad + HW + W + 8
    H = HW // W

    def body(*refs):
        it = iter(refs)
        xa_ref = next(it)
        if has_up:
            xlo_ref = next(it)
            wup_ref = next(it)
            bup_ref = next(it)
            w0b_ref = next(it)
        w0a_ref = next(it)
        b0_ref = next(it)
        w1_ref = next(it)
        b1_ref = next(it)
        w2_ref = next(it)
        b2_ref = next(it)
        w3_ref = next(it)
        b3_ref = next(it)
        w4_ref = next(it)
        b4_ref = next(it)
        o_ref = next(it)
        p_ref = next(it)

        def leaky(v):
            return jnp.maximum(v, _LEAKY * v)

        def mm(a, w):
            return jnp.dot(a.astype(jnp.bfloat16), w,
                           preferred_element_type=jnp.float32)

        acc0 = mm(xa_ref[0], w0a_ref[...])
        if has_up:
            zu = leaky(mm(xlo_ref[0], wup_ref[...]) + bup_ref[...])
            zlo = mm(zu, w0b_ref[...])
            h, w = H // 2, W // 2
            z3 = zlo.astype(jnp.bfloat16).astype(jnp.float32).reshape(h, w, -1)
            z3 = jnp.repeat(jnp.repeat(z3, 2, axis=1), 2, axis=0)
            acc0 = acc0 + z3.reshape(HW, -1)
        act = leaky(acc0 + b0_ref[...])

        col = jax.lax.broadcasted_iota(jnp.int32, (HW, 1), 0) % W
        not_l = col != 0
        not_r = col != (W - 1)

        C9 = p_ref.shape[1]
        C = C9 // 9

        p_ref[pad:pad + W + 1, 0:6 * C] = jnp.zeros((W + 1, 6 * C), jnp.bfloat16)
        p_ref[pad + HW - W - 1:pad + HW, 3 * C:C9] = (
            jnp.zeros((W + 1, 6 * C), jnp.bfloat16))

        def conv3(a, w_ref, b_ref):
            ab = a.astype(jnp.bfloat16)
            m = (jnp.where(not_r, ab, 0), ab, jnp.where(not_l, ab, 0))
            for dy in range(3):
                for dx in range(3):
                    ofs = pad - (dy - 1) * W + (1 - dx)
                    k = dy * 3 + dx
                    p_ref[ofs:ofs + HW, k * C:(k + 1) * C] = m[dx]
            acc = jnp.dot(p_ref[pad:pad + HW, :], w_ref[...],
                          preferred_element_type=jnp.float32)
            return leaky(acc + b_ref[...])

        z = conv3(act, w1_ref, b1_ref)
        z = leaky(mm(z, w2_ref[...]) + b2_ref[...])
        z = conv3(z, w3_ref, b3_ref)
        z = leaky(mm(z, w4_ref[...]) + b4_ref[...])
        o_ref[0] = z.astype(o_ref.dtype)

    return body


def _run_level(xa2, layers, H, W, up=None):
    N, HW, Ca = xa2.shape
    C = layers[4][0].shape[-1]
    has_up = up is not None
    pad = W + 8
    rows_p = pad + HW + W + 8

    def const_spec(shape):
        nd = len(shape)
        return pl.BlockSpec(shape, lambda n, _nd=nd: (0,) * _nd)

    inputs = [xa2]
    in_specs = [pl.BlockSpec((1, HW, Ca), lambda n: (n, 0, 0))]

    if has_up:
        xlo = up["xlo"]
        hw, Clo = xlo.shape[1], xlo.shape[2]
        w0 = layers[0][0]
        w0a, w0b = w0[:Ca], w0[Ca:]
        inputs += [xlo, up["w"], up["b"], w0b]
        in_specs += [pl.BlockSpec((1, hw, Clo), lambda n: (n, 0, 0)),
                     const_spec(up["w"].shape),
                     const_spec(up["b"].shape), const_spec(w0b.shape)]
    else:
        w0a = layers[0][0]

    inputs += [w0a, layers[0][1]]
    in_specs += [const_spec(w0a.shape), const_spec(layers[0][1].shape)]
    for w, b in layers[1:]:
        inputs += [w, b]
        in_specs += [const_spec(w.shape), const_spec(b.shape)]

    flops = 2 * HW * (Ca * C + 2 * (9 * C * 2 * C) + 2 * (2 * C * C))
    if has_up:
        flops += 2 * (hw * Clo * C + hw * C * C)
    flops *= N
    bytes_accessed = (sum(int(a.size) * a.dtype.itemsize for a in inputs)
                      + N * HW * C * 2)

    return pl.pallas_call(
        _make_level_body(HW, W, has_up),
        out_shape=jax.ShapeDtypeStruct((N, HW, C), jnp.bfloat16),
        grid_spec=pltpu.PrefetchScalarGridSpec(
            num_scalar_prefetch=0,
            grid=(N,),
            in_specs=in_specs,
            out_specs=pl.BlockSpec((1, HW, C), lambda n: (n, 0, 0)),
            scratch_shapes=[pltpu.VMEM((rows_p, 9 * C), jnp.bfloat16)],
        ),
        compiler_params=pltpu.CompilerParams(
            dimension_semantics=("parallel",),
            vmem_limit_bytes=_VMEM_LIMIT),
        cost_estimate=pl.CostEstimate(flops=flops, transcendentals=0,
                                      bytes_accessed=bytes_accessed),
    )(*inputs)


def _stack_taps(w9):
    k9, C, C2 = w9.shape
    return w9.reshape(9 * C, C2)


def kernel(x0, x1, x2,
           seq1_0_w, seq1_0_b, seq1_1_w, seq1_1_b, seq1_2_w, seq1_2_b,
           seq1_3_w, seq1_3_b, seq1_4_w, seq1_4_b,
           seq2_0_w, seq2_0_b, seq2_1_w, seq2_1_b, seq2_2_w, seq2_2_b,
           seq2_3_w, seq2_3_b, seq2_4_w, seq2_4_b,
           seq3_0_w, seq3_0_b, seq3_1_w, seq3_1_b, seq3_2_w, seq3_2_b,
           seq3_3_w, seq3_3_b, seq3_4_w, seq3_4_b,
           up1_w, up1_b, up2_w, up2_b):
    seqs = {
        1: [(seq1_0_w, seq1_0_b), (_stack_taps(seq1_1_w), seq1_1_b),
            (seq1_2_w, seq1_2_b), (_stack_taps(seq1_3_w), seq1_3_b),
            (seq1_4_w, seq1_4_b)],
        2: [(seq2_0_w, seq2_0_b), (_stack_taps(seq2_1_w), seq2_1_b),
            (seq2_2_w, seq2_2_b), (_stack_taps(seq2_3_w), seq2_3_b),
            (seq2_4_w, seq2_4_b)],
        3: [(seq3_0_w, seq3_0_b), (_stack_taps(seq3_1_w), seq3_1_b),
            (seq3_2_w, seq3_2_b), (_stack_taps(seq3_3_w), seq3_3_b),
            (seq3_4_w, seq3_4_b)],
    }

    feats, dims = [], []
    for x in (x0, x1, x2):
        n, c, h, w = x.shape
        feats.append(jnp.transpose(x, (0, 2, 3, 1)).astype(jnp.bfloat16)
                     .reshape(n, h * w, c))
        dims.append((h, w))
    (H0, W0), (H1, W1), (H2, W2) = dims

    n3 = _run_level(feats[2], seqs[3], H2, W2)
    n2 = _run_level(feats[1], seqs[2], H1, W1,
                    up=dict(xlo=n3, w=up2_w, b=up2_b))
    n1 = _run_level(feats[0], seqs[1], H0, W0,
                    up=dict(xlo=n2, w=up1_w, b=up1_b))

    def to_nchw(x2d, h, w):
        n, _, c = x2d.shape
        return jnp.transpose(x2d.reshape(n, h, w, c), (0, 3, 1, 2)).astype(jnp.float32)

    return [to_nchw(n1, H0, W0), to_nchw(n2, H1, W1), to_nchw(n3, H2, W2)]
